# baseline (speedup 1.0000x reference)
import functools

import jax
import jax.numpy as jnp
from jax import lax
from jax.experimental import pallas as pl
from jax.experimental.pallas import tpu as pltpu
from jax.experimental.pallas import tpu_sc as plsc

N, E, DIN, H, OUT, B = 10000, 320000, 128, 128, 64, 128

NP = 10240
BLK = 512
NBLK = NP // BLK
EK = 3456
ETOT = E + N
NTILES = 32
EP = ((ETOT + NTILES * EK - 1) // (NTILES * EK)) * (NTILES * EK)
EPT = EP // NTILES
ACH = EPT // EK
NCH = EP // EK
NGR = EK // 16
FPT = H // NTILES


def _tc_dense_body(x_ref, w_ref, asr_ref, adr_ref, h_ref, as_ref, ad_ref):
    xb = x_ref[...]
    hb = lax.dot_general(w_ref[...], xb, (((0,), (0,)), ((), ())),
                         preferred_element_type=jnp.float32)
    h_ref[...] = hb
    as_ref[...] = lax.dot_general(asr_ref[...], hb, (((1,), (0,)), ((), ())),
                                  preferred_element_type=jnp.float32)
    ad_ref[...] = lax.dot_general(adr_ref[...], hb, (((1,), (0,)), ((), ())),
                                  preferred_element_type=jnp.float32)


def _tc_dense(x_t, w, a_src, a_dst):
    return pl.pallas_call(
        _tc_dense_body,
        grid=(NBLK,),
        in_specs=[
            pl.BlockSpec((DIN, BLK), lambda j: (0, j)),
            pl.BlockSpec((DIN, H), lambda j: (0, 0)),
            pl.BlockSpec((1, H), lambda j: (0, 0)),
            pl.BlockSpec((1, H), lambda j: (0, 0)),
        ],
        out_specs=[
            pl.BlockSpec((H, BLK), lambda j: (0, j)),
            pl.BlockSpec((1, BLK), lambda j: (0, j)),
            pl.BlockSpec((1, BLK), lambda j: (0, j)),
        ],
        out_shape=[
            jax.ShapeDtypeStruct((H, NP), jnp.float32),
            jax.ShapeDtypeStruct((1, NP), jnp.float32),
            jax.ShapeDtypeStruct((1, NP), jnp.float32),
        ],
    )(x_t, w, a_src, a_dst)


def _tc_mid_body(acc_ref, denp_ref, b_ref, w_ref, asr_ref, adr_ref,
                 h_ref, as_ref, ad_ref):
    den = jnp.sum(denp_ref[...], axis=0, keepdims=True)
    g = acc_ref[...] / (den + 1e-30) + b_ref[...]
    g = jnp.maximum(g, 0.0)
    hb = lax.dot_general(w_ref[...], g, (((0,), (0,)), ((), ())),
                         preferred_element_type=jnp.float32)
    h_ref[...] = hb
    as_ref[...] = lax.dot_general(asr_ref[...], hb, (((1,), (0,)), ((), ())),
                                  preferred_element_type=jnp.float32)
    ad_ref[...] = lax.dot_general(adr_ref[...], hb, (((1,), (0,)), ((), ())),
                                  preferred_element_type=jnp.float32)


def _tc_mid(acc_t, den_p, b_col, w, a_src, a_dst):
    return pl.pallas_call(
        _tc_mid_body,
        grid=(NBLK,),
        in_specs=[
            pl.BlockSpec((H, BLK), lambda j: (0, j)),
            pl.BlockSpec((NTILES, BLK), lambda j: (0, j)),
            pl.BlockSpec((H, 1), lambda j: (0, 0)),
            pl.BlockSpec((H, H), lambda j: (0, 0)),
            pl.BlockSpec((1, H), lambda j: (0, 0)),
            pl.BlockSpec((1, H), lambda j: (0, 0)),
        ],
        out_specs=[
            pl.BlockSpec((H, BLK), lambda j: (0, j)),
            pl.BlockSpec((1, BLK), lambda j: (0, j)),
            pl.BlockSpec((1, BLK), lambda j: (0, j)),
        ],
        out_shape=[
            jax.ShapeDtypeStruct((H, NP), jnp.float32),
            jax.ShapeDtypeStruct((1, NP), jnp.float32),
            jax.ShapeDtypeStruct((1, NP), jnp.float32),
        ],
    )(acc_t, den_p, b_col, w, a_src, a_dst)


def _tc_final_body(acc_ref, denp_ref, b_ref, p_ref, wfc_ref, bfc_ref,
                   out_ref, pooled_ref, cnt_ref):
    j = pl.program_id(0)

    @pl.when(j == 0)
    def _():
        pooled_ref[...] = jnp.zeros_like(pooled_ref)
        cnt_ref[...] = jnp.zeros_like(cnt_ref)

    den = jnp.sum(denp_ref[...], axis=0, keepdims=True)
    g = acc_ref[...] / (den + 1e-30) + b_ref[...]
    g = jnp.maximum(g, 0.0)
    pb = p_ref[...]
    pooled_ref[...] += jnp.dot(g, pb, preferred_element_type=jnp.float32)
    cnt_ref[...] += jnp.sum(pb, axis=0, keepdims=True)

    @pl.when(j == NBLK - 1)
    def _():
        pooled = pooled_ref[...] / jnp.maximum(cnt_ref[...], 1.0)
        out_ref[...] = lax.dot_general(
            pooled, wfc_ref[...], (((0,), (0,)), ((), ())),
            preferred_element_type=jnp.float32) + bfc_ref[...]


def _tc_final(acc_t, den_p, b_col, p_mat, wfc, bfc_row):
    return pl.pallas_call(
        _tc_final_body,
        grid=(NBLK,),
        in_specs=[
            pl.BlockSpec((H, BLK), lambda j: (0, j)),
            pl.BlockSpec((NTILES, BLK), lambda j: (0, j)),
            pl.BlockSpec((H, 1), lambda j: (0, 0)),
            pl.BlockSpec((BLK, B), lambda j: (j, 0)),
            pl.BlockSpec((H, OUT), lambda j: (0, 0)),
            pl.BlockSpec((1, OUT), lambda j: (0, 0)),
        ],
        out_specs=pl.BlockSpec((B, OUT), lambda j: (0, 0)),
        out_shape=jax.ShapeDtypeStruct((B, OUT), jnp.float32),
        scratch_shapes=[
            pltpu.VMEM((H, B), jnp.float32),
            pltpu.VMEM((1, B), jnp.float32),
        ],
    )(acc_t, den_p, b_col, p_mat, wfc, bfc_row)


def _sc_mesh():
    return plsc.VectorSubcoreMesh(core_axis_name="c", subcore_axis_name="s",
                                  num_cores=2, num_subcores=16)


def _sc_pass_a_body(as_hbm, ad_hbm, src_hbm, dst_hbm, w_hbm, denp_hbm,
                    asv, adv, denv, sbuf, dbuf, wbuf):
    wid = lax.axis_index("c") * 16 + lax.axis_index("s")

    pltpu.sync_copy(as_hbm, asv)
    pltpu.sync_copy(ad_hbm, adv)

    neg = jnp.full((16,), -3.0e38, jnp.float32)

    def _mx(i, carry):
        ma, md = carry
        sl = pl.ds(i * 16, 16)
        return jnp.maximum(ma, asv[sl]), jnp.maximum(md, adv[sl])

    ma, md = lax.fori_loop(0, NP // 16, _mx, (neg, neg))

    def _hmax(v):
        idx = lax.iota(jnp.int32, 16)
        for k in (1, 2, 4, 8):
            perm = jnp.bitwise_xor(idx, k)
            g = lax.gather(
                v, perm[:, None],
                lax.GatherDimensionNumbers(
                    offset_dims=(), collapsed_slice_dims=(0,),
                    start_index_map=(0,)),
                (1,), mode=lax.GatherScatterMode.PROMISE_IN_BOUNDS)
            v = jnp.maximum(v, g)
        return v

    m_all = jnp.maximum(_hmax(ma) + _hmax(md), 0.0)

    zero = jnp.zeros((16,), jnp.float32)

    def _z(i, _):
        denv[pl.ds(i * 16, 16)] = zero
        return 0

    lax.fori_loop(0, NP // 16, _z, 0)

    for c in range(ACH):
        base = wid * EPT + c * EK
        pltpu.sync_copy(src_hbm.at[pl.ds(base, EK)], sbuf)
        pltpu.sync_copy(dst_hbm.at[pl.ds(base, EK)], dbuf)

        @plsc.parallel_loop(0, NGR, unroll=4)
        def _g(g):
            sl = pl.ds(g * 16, 16)
            s16 = sbuf[sl]
            d16 = dbuf[sl]
            z = plsc.load_gather(asv, [s16]) + plsc.load_gather(adv, [d16])
            e = jnp.where(z > 0.0, z, 0.2 * z)
            w16 = jnp.exp(e - m_all)
            wbuf[sl] = w16
            plsc.addupdate_scatter(denv, [d16], w16)
        pltpu.sync_copy(wbuf, w_hbm.at[pl.ds(base, EK)])

    pltpu.sync_copy(denv, denp_hbm.at[wid])


def _sc_pass_a(as_n, ad_n, src, dst):
    f = pl.kernel(
        _sc_pass_a_body,
        out_type=[
            jax.ShapeDtypeStruct((EP,), jnp.float32),
            jax.ShapeDtypeStruct((NTILES, NP), jnp.float32),
        ],
        mesh=_sc_mesh(),
        compiler_params=pltpu.CompilerParams(needs_layout_passes=False),
        scratch_types=[
            pltpu.VMEM((NP,), jnp.float32),
            pltpu.VMEM((NP,), jnp.float32),
            pltpu.VMEM((NP,), jnp.float32),
            pltpu.VMEM((EK,), jnp.int32),
            pltpu.VMEM((EK,), jnp.int32),
            pltpu.VMEM((EK,), jnp.float32),
        ],
    )
    return f(as_n, ad_n, src, dst)


def _sc_pass_b_body(h_hbm, src_hbm, dst_hbm, w_hbm, out_hbm,
                    hr0, hr1, hr2, hr3, ar0, ar1, ar2, ar3,
                    sbuf, dbuf, wbuf, sems):
    wid = lax.axis_index("c") * 16 + lax.axis_index("s")
    hrows = (hr0, hr1, hr2, hr3)
    arows = (ar0, ar1, ar2, ar3)

    for i in range(FPT):
        pltpu.sync_copy(h_hbm.at[wid * FPT + i], hrows[i])

    zero = jnp.zeros((16,), jnp.float32)

    def _z(i, _):
        for f in range(FPT):
            arows[f][pl.ds(i * 16, 16)] = zero
        return 0

    lax.fori_loop(0, NP // 16, _z, 0)

    def _start(c, slot):
        base = c * EK
        pltpu.async_copy(src_hbm.at[pl.ds(base, EK)], sbuf.at[slot],
                         sems.at[slot])
        pltpu.async_copy(dst_hbm.at[pl.ds(base, EK)], dbuf.at[slot],
                         sems.at[slot])
        pltpu.async_copy(w_hbm.at[pl.ds(base, EK)], wbuf.at[slot],
                         sems.at[slot])

    def _wait(c, slot):
        base = c * EK
        pltpu.make_async_copy(src_hbm.at[pl.ds(base, EK)], sbuf.at[slot],
                              sems.at[slot]).wait()
        pltpu.make_async_copy(dst_hbm.at[pl.ds(base, EK)], dbuf.at[slot],
                              sems.at[slot]).wait()
        pltpu.make_async_copy(w_hbm.at[pl.ds(base, EK)], wbuf.at[slot],
                              sems.at[slot]).wait()

    def _process(slot):
        @plsc.parallel_loop(0, NGR, unroll=8)
        def _g(g):
            sl = pl.ds(g * 16, 16)
            s16 = sbuf[slot, sl]
            d16 = dbuf[slot, sl]
            w16 = wbuf[slot, sl]
            vs = [plsc.load_gather(hrows[f], [s16]) for f in range(FPT)]
            for f in range(FPT):
                plsc.addupdate_scatter(arows[f], [d16], vs[f] * w16)

    _start(0, 0)

    def _outer(p, _):
        c0 = p * 2
        _wait(c0, 0)
        _start(c0 + 1, 1)
        _process(0)
        _wait(c0 + 1, 1)

        @pl.when(c0 + 2 < NCH)
        def _():
            _start(c0 + 2, 0)

        _process(1)
        return 0

    lax.fori_loop(0, NCH // 2, _outer, 0)

    for i in range(FPT):
        pltpu.sync_copy(arows[i], out_hbm.at[wid * FPT + i])


def _sc_pass_b(h_t, src, dst, w):
    f = pl.kernel(
        _sc_pass_b_body,
        out_type=jax.ShapeDtypeStruct((H, NP), jnp.float32),
        mesh=_sc_mesh(),
        compiler_params=pltpu.CompilerParams(needs_layout_passes=False),
        scratch_types=[
            pltpu.VMEM((NP,), jnp.float32),
            pltpu.VMEM((NP,), jnp.float32),
            pltpu.VMEM((NP,), jnp.float32),
            pltpu.VMEM((NP,), jnp.float32),
            pltpu.VMEM((NP,), jnp.float32),
            pltpu.VMEM((NP,), jnp.float32),
            pltpu.VMEM((NP,), jnp.float32),
            pltpu.VMEM((NP,), jnp.float32),
            pltpu.VMEM((2, EK), jnp.int32),
            pltpu.VMEM((2, EK), jnp.int32),
            pltpu.VMEM((2, EK), jnp.float32),
            pltpu.SemaphoreType.DMA((2,)),
        ],
    )
    return f(h_t, src, dst, w)


def kernel(x, edge_index, batch, W1, a_src1, a_dst1, b1,
           W2, a_src2, a_dst2, b2, Wfc, bfc):
    n = x.shape[0]
    loops = jnp.arange(n, dtype=edge_index.dtype)
    src = jnp.concatenate([edge_index[0], loops])
    dst = jnp.concatenate([edge_index[1], loops])
    src = jnp.concatenate(
        [src, jnp.zeros((EP - ETOT,), src.dtype)])
    dst = jnp.concatenate(
        [dst, jnp.full((EP - ETOT,), NP - 1, dst.dtype)])

    x_t = jnp.pad(x.T, ((0, 0), (0, NP - n)))

    h1, as1, ad1 = _tc_dense(x_t, W1, a_src1, a_dst1)
    w1, den1 = _sc_pass_a(as1.reshape(NP), ad1.reshape(NP), src, dst)
    acc1 = _sc_pass_b(h1, src, dst, w1)

    h2, as2, ad2 = _tc_mid(acc1, den1, b1.reshape(H, 1), W2, a_src2, a_dst2)
    w2, den2 = _sc_pass_a(as2.reshape(NP), ad2.reshape(NP), src, dst)
    acc2 = _sc_pass_b(h2, src, dst, w2)

    batch_pad = jnp.concatenate([batch, jnp.full((NP - n,), B, batch.dtype)])
    p_mat = (batch_pad[:, None] == jnp.arange(B, dtype=batch.dtype)[None, :]
             ).astype(jnp.float32)

    return _tc_final(acc2, den2, b2.reshape(H, 1), p_mat, Wfc,
                     bfc.reshape(1, OUT))


# speedup vs baseline: 1.0985x; 1.0985x over previous
import functools

import jax
import jax.numpy as jnp
from jax import lax
from jax.experimental import pallas as pl
from jax.experimental.pallas import tpu as pltpu
from jax.experimental.pallas import tpu_sc as plsc

N, E, DIN, H, OUT, B = 10000, 320000, 128, 128, 64, 128

NP = 10240
BLK = 512
NBLK = NP // BLK
EK = 3456
ETOT = E + N
NTILES = 32
EP = ((ETOT + NTILES * EK - 1) // (NTILES * EK)) * (NTILES * EK)
EPT = EP // NTILES
ACH = EPT // EK
NCH = EP // EK
NGR = EK // 16
FPT = H // NTILES


def _tc_dense_body(x_ref, w_ref, asr_ref, adr_ref, h_ref, as_ref, ad_ref):
    xb = x_ref[...]
    hb = lax.dot_general(w_ref[...], xb, (((0,), (0,)), ((), ())),
                         preferred_element_type=jnp.float32)
    h_ref[...] = hb
    as_ref[...] = lax.dot_general(asr_ref[...], hb, (((1,), (0,)), ((), ())),
                                  preferred_element_type=jnp.float32)
    ad_ref[...] = lax.dot_general(adr_ref[...], hb, (((1,), (0,)), ((), ())),
                                  preferred_element_type=jnp.float32)


def _tc_dense(x_t, w, a_src, a_dst):
    return pl.pallas_call(
        _tc_dense_body,
        grid=(NBLK,),
        in_specs=[
            pl.BlockSpec((DIN, BLK), lambda j: (0, j)),
            pl.BlockSpec((DIN, H), lambda j: (0, 0)),
            pl.BlockSpec((1, H), lambda j: (0, 0)),
            pl.BlockSpec((1, H), lambda j: (0, 0)),
        ],
        out_specs=[
            pl.BlockSpec((H, BLK), lambda j: (0, j)),
            pl.BlockSpec((1, BLK), lambda j: (0, j)),
            pl.BlockSpec((1, BLK), lambda j: (0, j)),
        ],
        out_shape=[
            jax.ShapeDtypeStruct((H, NP), jnp.float32),
            jax.ShapeDtypeStruct((1, NP), jnp.float32),
            jax.ShapeDtypeStruct((1, NP), jnp.float32),
        ],
    )(x_t, w, a_src, a_dst)


def _tc_mid_body(acc_ref, denp_ref, b_ref, w_ref, asr_ref, adr_ref,
                 h_ref, as_ref, ad_ref):
    den = jnp.sum(denp_ref[...], axis=0, keepdims=True)
    g = acc_ref[...] / (den + 1e-30) + b_ref[...]
    g = jnp.maximum(g, 0.0)
    hb = lax.dot_general(w_ref[...], g, (((0,), (0,)), ((), ())),
                         preferred_element_type=jnp.float32)
    h_ref[...] = hb
    as_ref[...] = lax.dot_general(asr_ref[...], hb, (((1,), (0,)), ((), ())),
                                  preferred_element_type=jnp.float32)
    ad_ref[...] = lax.dot_general(adr_ref[...], hb, (((1,), (0,)), ((), ())),
                                  preferred_element_type=jnp.float32)


def _tc_mid(acc_t, den_p, b_col, w, a_src, a_dst):
    return pl.pallas_call(
        _tc_mid_body,
        grid=(NBLK,),
        in_specs=[
            pl.BlockSpec((H, BLK), lambda j: (0, j)),
            pl.BlockSpec((NTILES, BLK), lambda j: (0, j)),
            pl.BlockSpec((H, 1), lambda j: (0, 0)),
            pl.BlockSpec((H, H), lambda j: (0, 0)),
            pl.BlockSpec((1, H), lambda j: (0, 0)),
            pl.BlockSpec((1, H), lambda j: (0, 0)),
        ],
        out_specs=[
            pl.BlockSpec((H, BLK), lambda j: (0, j)),
            pl.BlockSpec((1, BLK), lambda j: (0, j)),
            pl.BlockSpec((1, BLK), lambda j: (0, j)),
        ],
        out_shape=[
            jax.ShapeDtypeStruct((H, NP), jnp.float32),
            jax.ShapeDtypeStruct((1, NP), jnp.float32),
            jax.ShapeDtypeStruct((1, NP), jnp.float32),
        ],
    )(acc_t, den_p, b_col, w, a_src, a_dst)


def _tc_final_body(acc_ref, denp_ref, b_ref, p_ref, wfc_ref, bfc_ref,
                   out_ref, pooled_ref, cnt_ref):
    j = pl.program_id(0)

    @pl.when(j == 0)
    def _():
        pooled_ref[...] = jnp.zeros_like(pooled_ref)
        cnt_ref[...] = jnp.zeros_like(cnt_ref)

    den = jnp.sum(denp_ref[...], axis=0, keepdims=True)
    g = acc_ref[...] / (den + 1e-30) + b_ref[...]
    g = jnp.maximum(g, 0.0)
    pb = p_ref[...]
    pooled_ref[...] += jnp.dot(g, pb, preferred_element_type=jnp.float32)
    cnt_ref[...] += jnp.sum(pb, axis=0, keepdims=True)

    @pl.when(j == NBLK - 1)
    def _():
        pooled = pooled_ref[...] / jnp.maximum(cnt_ref[...], 1.0)
        out_ref[...] = lax.dot_general(
            pooled, wfc_ref[...], (((0,), (0,)), ((), ())),
            preferred_element_type=jnp.float32) + bfc_ref[...]


def _tc_final(acc_t, den_p, b_col, p_mat, wfc, bfc_row):
    return pl.pallas_call(
        _tc_final_body,
        grid=(NBLK,),
        in_specs=[
            pl.BlockSpec((H, BLK), lambda j: (0, j)),
            pl.BlockSpec((NTILES, BLK), lambda j: (0, j)),
            pl.BlockSpec((H, 1), lambda j: (0, 0)),
            pl.BlockSpec((BLK, B), lambda j: (j, 0)),
            pl.BlockSpec((H, OUT), lambda j: (0, 0)),
            pl.BlockSpec((1, OUT), lambda j: (0, 0)),
        ],
        out_specs=pl.BlockSpec((B, OUT), lambda j: (0, 0)),
        out_shape=jax.ShapeDtypeStruct((B, OUT), jnp.float32),
        scratch_shapes=[
            pltpu.VMEM((H, B), jnp.float32),
            pltpu.VMEM((1, B), jnp.float32),
        ],
    )(acc_t, den_p, b_col, p_mat, wfc, bfc_row)


def _sc_mesh():
    return plsc.VectorSubcoreMesh(core_axis_name="c", subcore_axis_name="s",
                                  num_cores=2, num_subcores=16)


def _sc_pass_a_body(as_hbm, ad_hbm, sd_hbm, w_hbm, denp_hbm,
                    asv, adv, denv, sdbuf, wbuf):
    wid = lax.axis_index("c") * 16 + lax.axis_index("s")

    pltpu.sync_copy(as_hbm, asv)
    pltpu.sync_copy(ad_hbm, adv)

    neg = jnp.full((16,), -3.0e38, jnp.float32)

    def _mx(i, carry):
        ma, md = carry
        sl = pl.ds(i * 16, 16)
        return jnp.maximum(ma, asv[sl]), jnp.maximum(md, adv[sl])

    ma, md = lax.fori_loop(0, NP // 16, _mx, (neg, neg))

    def _hmax(v):
        idx = lax.iota(jnp.int32, 16)
        for k in (1, 2, 4, 8):
            perm = jnp.bitwise_xor(idx, k)
            g = lax.gather(
                v, perm[:, None],
                lax.GatherDimensionNumbers(
                    offset_dims=(), collapsed_slice_dims=(0,),
                    start_index_map=(0,)),
                (1,), mode=lax.GatherScatterMode.PROMISE_IN_BOUNDS)
            v = jnp.maximum(v, g)
        return v

    m_all = jnp.maximum(_hmax(ma) + _hmax(md), 0.0)

    zero = jnp.zeros((16,), jnp.float32)

    def _z(i, _):
        denv[pl.ds(i * 16, 16)] = zero
        return 0

    lax.fori_loop(0, NP // 16, _z, 0)

    for c in range(ACH):
        base = wid * EPT + c * EK
        pltpu.sync_copy(sd_hbm.at[pl.ds(base, EK)], sdbuf)

        @plsc.parallel_loop(0, NGR, unroll=4)
        def _g(g):
            sl = pl.ds(g * 16, 16)
            sd16 = sdbuf[sl]
            s16 = jnp.bitwise_and(sd16, 0xFFFF)
            d16 = lax.shift_right_logical(sd16, 16)
            z = plsc.load_gather(asv, [s16]) + plsc.load_gather(adv, [d16])
            e = jnp.where(z > 0.0, z, 0.2 * z)
            w16 = jnp.exp(e - m_all)
            wbuf[sl] = w16
            plsc.addupdate_scatter(denv, [d16], w16)
        pltpu.sync_copy(wbuf, w_hbm.at[pl.ds(base, EK)])

    pltpu.sync_copy(denv, denp_hbm.at[wid])


def _sc_pass_a(as_n, ad_n, sd):
    f = pl.kernel(
        _sc_pass_a_body,
        out_type=[
            jax.ShapeDtypeStruct((EP,), jnp.float32),
            jax.ShapeDtypeStruct((NTILES, NP), jnp.float32),
        ],
        mesh=_sc_mesh(),
        compiler_params=pltpu.CompilerParams(needs_layout_passes=False),
        scratch_types=[
            pltpu.VMEM((NP,), jnp.float32),
            pltpu.VMEM((NP,), jnp.float32),
            pltpu.VMEM((NP,), jnp.float32),
            pltpu.VMEM((EK,), jnp.int32),
            pltpu.VMEM((EK,), jnp.float32),
        ],
    )
    return f(as_n, ad_n, sd)


def _sc_pass_b_body(h_hbm, sd_hbm, w_hbm, out_hbm,
                    hr0, hr1, hr2, hr3, ar0, ar1, ar2, ar3,
                    sdbuf, wbuf, sems):
    wid = lax.axis_index("c") * 16 + lax.axis_index("s")
    hrows = (hr0, hr1, hr2, hr3)
    arows = (ar0, ar1, ar2, ar3)

    for i in range(FPT):
        pltpu.sync_copy(h_hbm.at[wid * FPT + i], hrows[i])

    zero = jnp.zeros((16,), jnp.float32)

    def _z(i, _):
        for f in range(FPT):
            arows[f][pl.ds(i * 16, 16)] = zero
        return 0

    lax.fori_loop(0, NP // 16, _z, 0)

    def _start(c, slot):
        base = c * EK
        pltpu.async_copy(sd_hbm.at[pl.ds(base, EK)], sdbuf.at[slot],
                         sems.at[slot])
        pltpu.async_copy(w_hbm.at[pl.ds(base, EK)], wbuf.at[slot],
                         sems.at[slot])

    def _wait(c, slot):
        base = c * EK
        pltpu.make_async_copy(sd_hbm.at[pl.ds(base, EK)], sdbuf.at[slot],
                              sems.at[slot]).wait()
        pltpu.make_async_copy(w_hbm.at[pl.ds(base, EK)], wbuf.at[slot],
                              sems.at[slot]).wait()

    def _process(slot):
        @plsc.parallel_loop(0, NGR, unroll=4)
        def _g(g):
            sl = pl.ds(g * 16, 16)
            sd16 = sdbuf[slot, sl]
            s16 = jnp.bitwise_and(sd16, 0xFFFF)
            d16 = lax.shift_right_logical(sd16, 16)
            w16 = wbuf[slot, sl]
            vs = [plsc.load_gather(hrows[f], [s16]) for f in range(FPT)]
            for f in range(FPT):
                plsc.addupdate_scatter(arows[f], [d16], vs[f] * w16)

    _start(0, 0)

    def _outer(p, _):
        c0 = p * 2
        _wait(c0, 0)
        _start(c0 + 1, 1)
        _process(0)
        _wait(c0 + 1, 1)

        @pl.when(c0 + 2 < NCH)
        def _():
            _start(c0 + 2, 0)

        _process(1)
        return 0

    lax.fori_loop(0, NCH // 2, _outer, 0)

    for i in range(FPT):
        pltpu.sync_copy(arows[i], out_hbm.at[wid * FPT + i])


def _sc_pass_b(h_t, sd, w):
    f = pl.kernel(
        _sc_pass_b_body,
        out_type=jax.ShapeDtypeStruct((H, NP), jnp.float32),
        mesh=_sc_mesh(),
        compiler_params=pltpu.CompilerParams(needs_layout_passes=False),
        scratch_types=[
            pltpu.VMEM((NP,), jnp.float32),
            pltpu.VMEM((NP,), jnp.float32),
            pltpu.VMEM((NP,), jnp.float32),
            pltpu.VMEM((NP,), jnp.float32),
            pltpu.VMEM((NP,), jnp.float32),
            pltpu.VMEM((NP,), jnp.float32),
            pltpu.VMEM((NP,), jnp.float32),
            pltpu.VMEM((NP,), jnp.float32),
            pltpu.VMEM((2, EK), jnp.int32),
            pltpu.VMEM((2, EK), jnp.float32),
            pltpu.SemaphoreType.DMA((2,)),
        ],
    )
    return f(h_t, sd, w)


def kernel(x, edge_index, batch, W1, a_src1, a_dst1, b1,
           W2, a_src2, a_dst2, b2, Wfc, bfc):
    n = x.shape[0]
    loops = jnp.arange(n, dtype=edge_index.dtype)
    src = jnp.concatenate([edge_index[0], loops])
    dst = jnp.concatenate([edge_index[1], loops])
    src = jnp.concatenate(
        [src, jnp.zeros((EP - ETOT,), src.dtype)])
    dst = jnp.concatenate(
        [dst, jnp.full((EP - ETOT,), NP - 1, dst.dtype)])
    sd = jnp.bitwise_or(src, jnp.left_shift(dst, 16))

    x_t = jnp.pad(x.T, ((0, 0), (0, NP - n)))

    h1, as1, ad1 = _tc_dense(x_t, W1, a_src1, a_dst1)
    w1, den1 = _sc_pass_a(as1.reshape(NP), ad1.reshape(NP), sd)
    acc1 = _sc_pass_b(h1, sd, w1)

    h2, as2, ad2 = _tc_mid(acc1, den1, b1.reshape(H, 1), W2, a_src2, a_dst2)
    w2, den2 = _sc_pass_a(as2.reshape(NP), ad2.reshape(NP), sd)
    acc2 = _sc_pass_b(h2, sd, w2)

    batch_pad = jnp.concatenate([batch, jnp.full((NP - n,), B, batch.dtype)])
    p_mat = (batch_pad[:, None] == jnp.arange(B, dtype=batch.dtype)[None, :]
             ).astype(jnp.float32)

    return _tc_final(acc2, den2, b2.reshape(H, 1), p_mat, Wfc,
                     bfc.reshape(1, OUT))


# speedup vs baseline: 1.2876x; 1.1721x over previous
import functools

import jax
import jax.numpy as jnp
from jax import lax
from jax.experimental import pallas as pl
from jax.experimental.pallas import tpu as pltpu
from jax.experimental.pallas import tpu_sc as plsc

N, E, DIN, H, OUT, B = 10000, 320000, 128, 128, 64, 128

NP = 10240
BLK = 512
NBLK = NP // BLK
EK = 3456
ETOT = E + N
NTILES = 32
EP = ((ETOT + NTILES * EK - 1) // (NTILES * EK)) * (NTILES * EK)
EPT = EP // NTILES
ACH = EPT // EK
NCH = EP // EK
NGR = EK // 16
FPT = H // NTILES


def _pack_pairs(hb):
    lo = lax.bitcast_convert_type(hb[:H // 2].astype(jnp.bfloat16),
                                  jnp.uint16).astype(jnp.uint32)
    hi = lax.bitcast_convert_type(hb[H // 2:].astype(jnp.bfloat16),
                                  jnp.uint16).astype(jnp.uint32)
    return lax.bitcast_convert_type(
        jnp.bitwise_or(lo, jnp.left_shift(hi, 16)), jnp.int32)


def _tc_dense_body(x_ref, w_ref, asr_ref, adr_ref, h_ref, as_ref, ad_ref):
    xb = x_ref[...]
    hb = lax.dot_general(w_ref[...], xb, (((0,), (0,)), ((), ())),
                         preferred_element_type=jnp.float32)
    h_ref[...] = _pack_pairs(hb)
    as_ref[...] = lax.dot_general(asr_ref[...], hb, (((1,), (0,)), ((), ())),
                                  preferred_element_type=jnp.float32)
    ad_ref[...] = lax.dot_general(adr_ref[...], hb, (((1,), (0,)), ((), ())),
                                  preferred_element_type=jnp.float32)


def _tc_dense(x_t, w, a_src, a_dst):
    return pl.pallas_call(
        _tc_dense_body,
        grid=(NBLK,),
        in_specs=[
            pl.BlockSpec((DIN, BLK), lambda j: (0, j)),
            pl.BlockSpec((DIN, H), lambda j: (0, 0)),
            pl.BlockSpec((1, H), lambda j: (0, 0)),
            pl.BlockSpec((1, H), lambda j: (0, 0)),
        ],
        out_specs=[
            pl.BlockSpec((H // 2, BLK), lambda j: (0, j)),
            pl.BlockSpec((1, BLK), lambda j: (0, j)),
            pl.BlockSpec((1, BLK), lambda j: (0, j)),
        ],
        out_shape=[
            jax.ShapeDtypeStruct((H // 2, NP), jnp.int32),
            jax.ShapeDtypeStruct((1, NP), jnp.float32),
            jax.ShapeDtypeStruct((1, NP), jnp.float32),
        ],
    )(x_t, w, a_src, a_dst)


def _tc_mid_body(acc_ref, denp_ref, b_ref, w_ref, asr_ref, adr_ref,
                 h_ref, as_ref, ad_ref):
    den = jnp.sum(denp_ref[...], axis=0, keepdims=True)
    g = acc_ref[...] / (den + 1e-30) + b_ref[...]
    g = jnp.maximum(g, 0.0)
    hb = lax.dot_general(w_ref[...], g, (((0,), (0,)), ((), ())),
                         preferred_element_type=jnp.float32)
    h_ref[...] = _pack_pairs(hb)
    as_ref[...] = lax.dot_general(asr_ref[...], hb, (((1,), (0,)), ((), ())),
                                  preferred_element_type=jnp.float32)
    ad_ref[...] = lax.dot_general(adr_ref[...], hb, (((1,), (0,)), ((), ())),
                                  preferred_element_type=jnp.float32)


def _tc_mid(acc_t, den_p, b_col, w, a_src, a_dst):
    return pl.pallas_call(
        _tc_mid_body,
        grid=(NBLK,),
        in_specs=[
            pl.BlockSpec((H, BLK), lambda j: (0, j)),
            pl.BlockSpec((NTILES, BLK), lambda j: (0, j)),
            pl.BlockSpec((H, 1), lambda j: (0, 0)),
            pl.BlockSpec((H, H), lambda j: (0, 0)),
            pl.BlockSpec((1, H), lambda j: (0, 0)),
            pl.BlockSpec((1, H), lambda j: (0, 0)),
        ],
        out_specs=[
            pl.BlockSpec((H // 2, BLK), lambda j: (0, j)),
            pl.BlockSpec((1, BLK), lambda j: (0, j)),
            pl.BlockSpec((1, BLK), lambda j: (0, j)),
        ],
        out_shape=[
            jax.ShapeDtypeStruct((H // 2, NP), jnp.int32),
            jax.ShapeDtypeStruct((1, NP), jnp.float32),
            jax.ShapeDtypeStruct((1, NP), jnp.float32),
        ],
    )(acc_t, den_p, b_col, w, a_src, a_dst)


def _tc_final_body(acc_ref, denp_ref, b_ref, p_ref, wfc_ref, bfc_ref,
                   out_ref, pooled_ref, cnt_ref):
    j = pl.program_id(0)

    @pl.when(j == 0)
    def _():
        pooled_ref[...] = jnp.zeros_like(pooled_ref)
        cnt_ref[...] = jnp.zeros_like(cnt_ref)

    den = jnp.sum(denp_ref[...], axis=0, keepdims=True)
    g = acc_ref[...] / (den + 1e-30) + b_ref[...]
    g = jnp.maximum(g, 0.0)
    pb = p_ref[...]
    pooled_ref[...] += jnp.dot(g, pb, preferred_element_type=jnp.float32)
    cnt_ref[...] += jnp.sum(pb, axis=0, keepdims=True)

    @pl.when(j == NBLK - 1)
    def _():
        pooled = pooled_ref[...] / jnp.maximum(cnt_ref[...], 1.0)
        out_ref[...] = lax.dot_general(
            pooled, wfc_ref[...], (((0,), (0,)), ((), ())),
            preferred_element_type=jnp.float32) + bfc_ref[...]


def _tc_final(acc_t, den_p, b_col, p_mat, wfc, bfc_row):
    return pl.pallas_call(
        _tc_final_body,
        grid=(NBLK,),
        in_specs=[
            pl.BlockSpec((H, BLK), lambda j: (0, j)),
            pl.BlockSpec((NTILES, BLK), lambda j: (0, j)),
            pl.BlockSpec((H, 1), lambda j: (0, 0)),
            pl.BlockSpec((BLK, B), lambda j: (j, 0)),
            pl.BlockSpec((H, OUT), lambda j: (0, 0)),
            pl.BlockSpec((1, OUT), lambda j: (0, 0)),
        ],
        out_specs=pl.BlockSpec((B, OUT), lambda j: (0, 0)),
        out_shape=jax.ShapeDtypeStruct((B, OUT), jnp.float32),
        scratch_shapes=[
            pltpu.VMEM((H, B), jnp.float32),
            pltpu.VMEM((1, B), jnp.float32),
        ],
    )(acc_t, den_p, b_col, p_mat, wfc, bfc_row)


def _sc_mesh():
    return plsc.VectorSubcoreMesh(core_axis_name="c", subcore_axis_name="s",
                                  num_cores=2, num_subcores=16)


def _sc_pass_a_body(as_hbm, ad_hbm, sd_hbm, w_hbm, denp_hbm,
                    asv, adv, denv, sdbuf, wbuf):
    wid = lax.axis_index("c") * 16 + lax.axis_index("s")

    pltpu.sync_copy(as_hbm, asv)
    pltpu.sync_copy(ad_hbm, adv)

    neg = jnp.full((16,), -3.0e38, jnp.float32)

    def _mx(i, carry):
        ma, md = carry
        sl = pl.ds(i * 16, 16)
        return jnp.maximum(ma, asv[sl]), jnp.maximum(md, adv[sl])

    ma, md = lax.fori_loop(0, NP // 16, _mx, (neg, neg))

    def _hmax(v):
        idx = lax.iota(jnp.int32, 16)
        for k in (1, 2, 4, 8):
            perm = jnp.bitwise_xor(idx, k)
            g = lax.gather(
                v, perm[:, None],
                lax.GatherDimensionNumbers(
                    offset_dims=(), collapsed_slice_dims=(0,),
                    start_index_map=(0,)),
                (1,), mode=lax.GatherScatterMode.PROMISE_IN_BOUNDS)
            v = jnp.maximum(v, g)
        return v

    m_all = jnp.maximum(_hmax(ma) + _hmax(md), 0.0)

    zero = jnp.zeros((16,), jnp.float32)

    def _z(i, _):
        denv[pl.ds(i * 16, 16)] = zero
        return 0

    lax.fori_loop(0, NP // 16, _z, 0)

    for c in range(ACH):
        base = wid * EPT + c * EK
        pltpu.sync_copy(sd_hbm.at[pl.ds(base, EK)], sdbuf)

        @plsc.parallel_loop(0, NGR, unroll=4)
        def _g(g):
            sl = pl.ds(g * 16, 16)
            sd16 = sdbuf[sl]
            s16 = jnp.bitwise_and(sd16, 0xFFFF)
            d16 = lax.shift_right_logical(sd16, 16)
            z = plsc.load_gather(asv, [s16]) + plsc.load_gather(adv, [d16])
            e = jnp.where(z > 0.0, z, 0.2 * z)
            w16 = jnp.exp(e - m_all)
            wbuf[sl] = w16
            plsc.addupdate_scatter(denv, [d16], w16)
        pltpu.sync_copy(wbuf, w_hbm.at[pl.ds(base, EK)])

    pltpu.sync_copy(denv, denp_hbm.at[wid])


def _sc_pass_a(as_n, ad_n, sd):
    f = pl.kernel(
        _sc_pass_a_body,
        out_type=[
            jax.ShapeDtypeStruct((EP,), jnp.float32),
            jax.ShapeDtypeStruct((NTILES, NP), jnp.float32),
        ],
        mesh=_sc_mesh(),
        compiler_params=pltpu.CompilerParams(needs_layout_passes=False),
        scratch_types=[
            pltpu.VMEM((NP,), jnp.float32),
            pltpu.VMEM((NP,), jnp.float32),
            pltpu.VMEM((NP,), jnp.float32),
            pltpu.VMEM((EK,), jnp.int32),
            pltpu.VMEM((EK,), jnp.float32),
        ],
    )
    return f(as_n, ad_n, sd)


def _sc_pass_b_body(h_hbm, sd_hbm, w_hbm, out_hbm,
                    hp0, hp1, ar0, ar1, ar2, ar3,
                    sdbuf, wbuf, sems):
    wid = lax.axis_index("c") * 16 + lax.axis_index("s")
    hprows = (hp0, hp1)
    arows = (ar0, ar1, ar2, ar3)

    for i in range(2):
        pltpu.sync_copy(h_hbm.at[wid * 2 + i], hprows[i])

    zero = jnp.zeros((16,), jnp.float32)

    def _z(i, _):
        for f in range(FPT):
            arows[f][pl.ds(i * 16, 16)] = zero
        return 0

    lax.fori_loop(0, NP // 16, _z, 0)

    def _start(c, slot):
        base = c * EK
        pltpu.async_copy(sd_hbm.at[pl.ds(base, EK)], sdbuf.at[slot],
                         sems.at[slot])
        pltpu.async_copy(w_hbm.at[pl.ds(base, EK)], wbuf.at[slot],
                         sems.at[slot])

    def _wait(c, slot):
        base = c * EK
        pltpu.make_async_copy(sd_hbm.at[pl.ds(base, EK)], sdbuf.at[slot],
                              sems.at[slot]).wait()
        pltpu.make_async_copy(w_hbm.at[pl.ds(base, EK)], wbuf.at[slot],
                              sems.at[slot]).wait()

    def _process(slot):
        @plsc.parallel_loop(0, NGR, unroll=4)
        def _g(g):
            sl = pl.ds(g * 16, 16)
            sd16 = sdbuf[slot, sl]
            s16 = jnp.bitwise_and(sd16, 0xFFFF)
            d16 = lax.shift_right_logical(sd16, 16)
            w16 = wbuf[slot, sl]
            for p in range(2):
                vp = plsc.load_gather(hprows[p], [s16])
                bb = plsc.bitcast(vp, jnp.bfloat16)
                lo, hi = plsc.unpack(bb, format=plsc.PackFormat.INTERLEAVED)
                plsc.addupdate_scatter(arows[p], [d16], lo * w16)
                plsc.addupdate_scatter(arows[2 + p], [d16], hi * w16)

    _start(0, 0)

    def _outer(p, _):
        c0 = p * 2
        _wait(c0, 0)
        _start(c0 + 1, 1)
        _process(0)
        _wait(c0 + 1, 1)

        @pl.when(c0 + 2 < NCH)
        def _():
            _start(c0 + 2, 0)

        _process(1)
        return 0

    lax.fori_loop(0, NCH // 2, _outer, 0)

    pltpu.sync_copy(arows[0], out_hbm.at[wid * 2])
    pltpu.sync_copy(arows[1], out_hbm.at[wid * 2 + 1])
    pltpu.sync_copy(arows[2], out_hbm.at[wid * 2 + H // 2])
    pltpu.sync_copy(arows[3], out_hbm.at[wid * 2 + 1 + H // 2])


def _sc_pass_b(h_t, sd, w):
    f = pl.kernel(
        _sc_pass_b_body,
        out_type=jax.ShapeDtypeStruct((H, NP), jnp.float32),
        mesh=_sc_mesh(),
        compiler_params=pltpu.CompilerParams(needs_layout_passes=False),
        scratch_types=[
            pltpu.VMEM((NP,), jnp.int32),
            pltpu.VMEM((NP,), jnp.int32),
            pltpu.VMEM((NP,), jnp.float32),
            pltpu.VMEM((NP,), jnp.float32),
            pltpu.VMEM((NP,), jnp.float32),
            pltpu.VMEM((NP,), jnp.float32),
            pltpu.VMEM((2, EK), jnp.int32),
            pltpu.VMEM((2, EK), jnp.float32),
            pltpu.SemaphoreType.DMA((2,)),
        ],
    )
    return f(h_t, sd, w)


def kernel(x, edge_index, batch, W1, a_src1, a_dst1, b1,
           W2, a_src2, a_dst2, b2, Wfc, bfc):
    n = x.shape[0]
    loops = jnp.arange(n, dtype=edge_index.dtype)
    src = jnp.concatenate([edge_index[0], loops])
    dst = jnp.concatenate([edge_index[1], loops])
    src = jnp.concatenate(
        [src, jnp.zeros((EP - ETOT,), src.dtype)])
    dst = jnp.concatenate(
        [dst, jnp.full((EP - ETOT,), NP - 1, dst.dtype)])
    sd = jnp.bitwise_or(src, jnp.left_shift(dst, 16))

    x_t = jnp.pad(x.T, ((0, 0), (0, NP - n)))

    h1, as1, ad1 = _tc_dense(x_t, W1, a_src1, a_dst1)
    w1, den1 = _sc_pass_a(as1.reshape(NP), ad1.reshape(NP), sd)
    acc1 = _sc_pass_b(h1, sd, w1)

    h2, as2, ad2 = _tc_mid(acc1, den1, b1.reshape(H, 1), W2, a_src2, a_dst2)
    w2, den2 = _sc_pass_a(as2.reshape(NP), ad2.reshape(NP), sd)
    acc2 = _sc_pass_b(h2, sd, w2)

    batch_pad = jnp.concatenate([batch, jnp.full((NP - n,), B, batch.dtype)])
    p_mat = (batch_pad[:, None] == jnp.arange(B, dtype=batch.dtype)[None, :]
             ).astype(jnp.float32)

    return _tc_final(acc2, den2, b2.reshape(H, 1), p_mat, Wfc,
                     bfc.reshape(1, OUT))


# speedup vs baseline: 1.3356x; 1.0373x over previous
import functools

import jax
import jax.numpy as jnp
from jax import lax
from jax.experimental import pallas as pl
from jax.experimental.pallas import tpu as pltpu
from jax.experimental.pallas import tpu_sc as plsc

N, E, DIN, H, OUT, B = 10000, 320000, 128, 128, 64, 128

NP = 10240
BLK = 512
NBLK = NP // BLK
EK = 3456
ETOT = E + N
NTILES = 32
EP = ((ETOT + NTILES * EK - 1) // (NTILES * EK)) * (NTILES * EK)
EPT = EP // NTILES
ACH = EPT // EK
NCH = EP // EK
NGR = EK // 16
FPT = H // NTILES


def _pack_pairs(hb):
    lo = lax.bitcast_convert_type(hb[:H // 2].astype(jnp.bfloat16),
                                  jnp.uint16).astype(jnp.uint32)
    hi = lax.bitcast_convert_type(hb[H // 2:].astype(jnp.bfloat16),
                                  jnp.uint16).astype(jnp.uint32)
    return lax.bitcast_convert_type(
        jnp.bitwise_or(lo, jnp.left_shift(hi, 16)), jnp.int32)


def _tc_dense_body(x_ref, w_ref, asr_ref, adr_ref, h_ref, as_ref, ad_ref):
    xb = x_ref[...]
    hb = lax.dot_general(w_ref[...], xb, (((0,), (1,)), ((), ())),
                         preferred_element_type=jnp.float32)
    h_ref[...] = _pack_pairs(hb)
    as_ref[...] = lax.dot_general(asr_ref[...], hb, (((1,), (0,)), ((), ())),
                                  preferred_element_type=jnp.float32)
    ad_ref[...] = lax.dot_general(adr_ref[...], hb, (((1,), (0,)), ((), ())),
                                  preferred_element_type=jnp.float32)


def _tc_dense(x_t, w, a_src, a_dst):
    return pl.pallas_call(
        _tc_dense_body,
        grid=(NBLK,),
        in_specs=[
            pl.BlockSpec((BLK, DIN), lambda j: (j, 0)),
            pl.BlockSpec((DIN, H), lambda j: (0, 0)),
            pl.BlockSpec((1, H), lambda j: (0, 0)),
            pl.BlockSpec((1, H), lambda j: (0, 0)),
        ],
        out_specs=[
            pl.BlockSpec((H // 2, BLK), lambda j: (0, j)),
            pl.BlockSpec((1, BLK), lambda j: (0, j)),
            pl.BlockSpec((1, BLK), lambda j: (0, j)),
        ],
        out_shape=[
            jax.ShapeDtypeStruct((H // 2, NP), jnp.int32),
            jax.ShapeDtypeStruct((1, NP), jnp.float32),
            jax.ShapeDtypeStruct((1, NP), jnp.float32),
        ],
    )(x_t, w, a_src, a_dst)


def _tc_mid_body(acc_ref, denp_ref, b_ref, w_ref, asr_ref, adr_ref,
                 h_ref, as_ref, ad_ref):
    den = jnp.sum(denp_ref[...], axis=0, keepdims=True)
    g = acc_ref[...] / (den + 1e-30) + b_ref[...]
    g = jnp.maximum(g, 0.0)
    hb = lax.dot_general(w_ref[...], g, (((0,), (0,)), ((), ())),
                         preferred_element_type=jnp.float32)
    h_ref[...] = _pack_pairs(hb)
    as_ref[...] = lax.dot_general(asr_ref[...], hb, (((1,), (0,)), ((), ())),
                                  preferred_element_type=jnp.float32)
    ad_ref[...] = lax.dot_general(adr_ref[...], hb, (((1,), (0,)), ((), ())),
                                  preferred_element_type=jnp.float32)


def _tc_mid(acc_t, den_p, b_col, w, a_src, a_dst):
    return pl.pallas_call(
        _tc_mid_body,
        grid=(NBLK,),
        in_specs=[
            pl.BlockSpec((H, BLK), lambda j: (0, j)),
            pl.BlockSpec((NTILES, BLK), lambda j: (0, j)),
            pl.BlockSpec((H, 1), lambda j: (0, 0)),
            pl.BlockSpec((H, H), lambda j: (0, 0)),
            pl.BlockSpec((1, H), lambda j: (0, 0)),
            pl.BlockSpec((1, H), lambda j: (0, 0)),
        ],
        out_specs=[
            pl.BlockSpec((H // 2, BLK), lambda j: (0, j)),
            pl.BlockSpec((1, BLK), lambda j: (0, j)),
            pl.BlockSpec((1, BLK), lambda j: (0, j)),
        ],
        out_shape=[
            jax.ShapeDtypeStruct((H // 2, NP), jnp.int32),
            jax.ShapeDtypeStruct((1, NP), jnp.float32),
            jax.ShapeDtypeStruct((1, NP), jnp.float32),
        ],
    )(acc_t, den_p, b_col, w, a_src, a_dst)


def _tc_final_body(acc_ref, denp_ref, b_ref, batch_ref, wfc_ref, bfc_ref,
                   out_ref, pooled_ref, cnt_ref):
    j = pl.program_id(0)

    @pl.when(j == 0)
    def _():
        pooled_ref[...] = jnp.zeros_like(pooled_ref)
        cnt_ref[...] = jnp.zeros_like(cnt_ref)

    den = jnp.sum(denp_ref[...], axis=0, keepdims=True)
    g = acc_ref[...] / (den + 1e-30) + b_ref[...]
    g = jnp.maximum(g, 0.0)
    ids = lax.broadcasted_iota(jnp.int32, (BLK, B), 1)
    pb = (batch_ref[...] == ids).astype(jnp.float32)
    pooled_ref[...] += jnp.dot(g, pb, preferred_element_type=jnp.float32)
    cnt_ref[...] += jnp.sum(pb, axis=0, keepdims=True)

    @pl.when(j == NBLK - 1)
    def _():
        pooled = pooled_ref[...] / jnp.maximum(cnt_ref[...], 1.0)
        out_ref[...] = lax.dot_general(
            pooled, wfc_ref[...], (((0,), (0,)), ((), ())),
            preferred_element_type=jnp.float32) + bfc_ref[...]


def _tc_final(acc_t, den_p, b_col, batch_col, wfc, bfc_row):
    return pl.pallas_call(
        _tc_final_body,
        grid=(NBLK,),
        in_specs=[
            pl.BlockSpec((H, BLK), lambda j: (0, j)),
            pl.BlockSpec((NTILES, BLK), lambda j: (0, j)),
            pl.BlockSpec((H, 1), lambda j: (0, 0)),
            pl.BlockSpec((BLK, 1), lambda j: (j, 0)),
            pl.BlockSpec((H, OUT), lambda j: (0, 0)),
            pl.BlockSpec((1, OUT), lambda j: (0, 0)),
        ],
        out_specs=pl.BlockSpec((B, OUT), lambda j: (0, 0)),
        out_shape=jax.ShapeDtypeStruct((B, OUT), jnp.float32),
        scratch_shapes=[
            pltpu.VMEM((H, B), jnp.float32),
            pltpu.VMEM((1, B), jnp.float32),
        ],
    )(acc_t, den_p, b_col, batch_col, wfc, bfc_row)


def _sc_mesh():
    return plsc.VectorSubcoreMesh(core_axis_name="c", subcore_axis_name="s",
                                  num_cores=2, num_subcores=16)


def _sc_pass_a_body(as_hbm, ad_hbm, sd_hbm, w_hbm, denp_hbm,
                    asv, adv, denv, sdbuf, wbuf):
    wid = lax.axis_index("c") * 16 + lax.axis_index("s")

    pltpu.sync_copy(as_hbm, asv)
    pltpu.sync_copy(ad_hbm, adv)

    neg = jnp.full((16,), -3.0e38, jnp.float32)

    def _mx(i, carry):
        ma, md = carry
        sl = pl.ds(i * 16, 16)
        return jnp.maximum(ma, asv[sl]), jnp.maximum(md, adv[sl])

    ma, md = lax.fori_loop(0, NP // 16, _mx, (neg, neg))

    def _hmax(v):
        idx = lax.iota(jnp.int32, 16)
        for k in (1, 2, 4, 8):
            perm = jnp.bitwise_xor(idx, k)
            g = lax.gather(
                v, perm[:, None],
                lax.GatherDimensionNumbers(
                    offset_dims=(), collapsed_slice_dims=(0,),
                    start_index_map=(0,)),
                (1,), mode=lax.GatherScatterMode.PROMISE_IN_BOUNDS)
            v = jnp.maximum(v, g)
        return v

    m_all = jnp.maximum(_hmax(ma) + _hmax(md), 0.0)

    zero = jnp.zeros((16,), jnp.float32)

    def _z(i, _):
        denv[pl.ds(i * 16, 16)] = zero
        return 0

    lax.fori_loop(0, NP // 16, _z, 0)

    for c in range(ACH):
        base = wid * EPT + c * EK
        pltpu.sync_copy(sd_hbm.at[pl.ds(base, EK)], sdbuf)

        @plsc.parallel_loop(0, NGR, unroll=4)
        def _g(g):
            sl = pl.ds(g * 16, 16)
            sd16 = sdbuf[sl]
            s16 = jnp.bitwise_and(sd16, 0xFFFF)
            d16 = lax.shift_right_logical(sd16, 16)
            z = plsc.load_gather(asv, [s16]) + plsc.load_gather(adv, [d16])
            e = jnp.where(z > 0.0, z, 0.2 * z)
            w16 = jnp.exp(e - m_all)
            wbuf[sl] = w16
            plsc.addupdate_scatter(denv, [d16], w16)
        pltpu.sync_copy(wbuf, w_hbm.at[pl.ds(base, EK)])

    pltpu.sync_copy(denv, denp_hbm.at[wid])


def _sc_pass_a(as_n, ad_n, sd):
    f = pl.kernel(
        _sc_pass_a_body,
        out_type=[
            jax.ShapeDtypeStruct((EP,), jnp.float32),
            jax.ShapeDtypeStruct((NTILES, NP), jnp.float32),
        ],
        mesh=_sc_mesh(),
        compiler_params=pltpu.CompilerParams(needs_layout_passes=False),
        scratch_types=[
            pltpu.VMEM((NP,), jnp.float32),
            pltpu.VMEM((NP,), jnp.float32),
            pltpu.VMEM((NP,), jnp.float32),
            pltpu.VMEM((EK,), jnp.int32),
            pltpu.VMEM((EK,), jnp.float32),
        ],
    )
    return f(as_n, ad_n, sd)


def _sc_pass_b_body(h_hbm, sd_hbm, w_hbm, out_hbm,
                    hp0, hp1, ar0, ar1, ar2, ar3,
                    sdbuf, wbuf, sems):
    wid = lax.axis_index("c") * 16 + lax.axis_index("s")
    hprows = (hp0, hp1)
    arows = (ar0, ar1, ar2, ar3)

    for i in range(2):
        pltpu.sync_copy(h_hbm.at[wid * 2 + i], hprows[i])

    zero = jnp.zeros((16,), jnp.float32)

    def _z(i, _):
        for f in range(FPT):
            arows[f][pl.ds(i * 16, 16)] = zero
        return 0

    lax.fori_loop(0, NP // 16, _z, 0)

    def _start(c, slot):
        base = c * EK
        pltpu.async_copy(sd_hbm.at[pl.ds(base, EK)], sdbuf.at[slot],
                         sems.at[slot])
        pltpu.async_copy(w_hbm.at[pl.ds(base, EK)], wbuf.at[slot],
                         sems.at[slot])

    def _wait(c, slot):
        base = c * EK
        pltpu.make_async_copy(sd_hbm.at[pl.ds(base, EK)], sdbuf.at[slot],
                              sems.at[slot]).wait()
        pltpu.make_async_copy(w_hbm.at[pl.ds(base, EK)], wbuf.at[slot],
                              sems.at[slot]).wait()

    def _process(slot):
        @plsc.parallel_loop(0, NGR, unroll=4)
        def _g(g):
            sl = pl.ds(g * 16, 16)
            sd16 = sdbuf[slot, sl]
            s16 = jnp.bitwise_and(sd16, 0xFFFF)
            d16 = lax.shift_right_logical(sd16, 16)
            w16 = wbuf[slot, sl]
            for p in range(2):
                vp = plsc.load_gather(hprows[p], [s16])
                bb = plsc.bitcast(vp, jnp.bfloat16)
                lo, hi = plsc.unpack(bb, format=plsc.PackFormat.INTERLEAVED)
                plsc.addupdate_scatter(arows[p], [d16], lo * w16)
                plsc.addupdate_scatter(arows[2 + p], [d16], hi * w16)

    _start(0, 0)

    def _outer(p, _):
        c0 = p * 2
        _wait(c0, 0)
        _start(c0 + 1, 1)
        _process(0)
        _wait(c0 + 1, 1)

        @pl.when(c0 + 2 < NCH)
        def _():
            _start(c0 + 2, 0)

        _process(1)
        return 0

    lax.fori_loop(0, NCH // 2, _outer, 0)

    pltpu.sync_copy(arows[0], out_hbm.at[wid * 2])
    pltpu.sync_copy(arows[1], out_hbm.at[wid * 2 + 1])
    pltpu.sync_copy(arows[2], out_hbm.at[wid * 2 + H // 2])
    pltpu.sync_copy(arows[3], out_hbm.at[wid * 2 + 1 + H // 2])


def _sc_pass_b(h_t, sd, w):
    f = pl.kernel(
        _sc_pass_b_body,
        out_type=jax.ShapeDtypeStruct((H, NP), jnp.float32),
        mesh=_sc_mesh(),
        compiler_params=pltpu.CompilerParams(needs_layout_passes=False),
        scratch_types=[
            pltpu.VMEM((NP,), jnp.int32),
            pltpu.VMEM((NP,), jnp.int32),
            pltpu.VMEM((NP,), jnp.float32),
            pltpu.VMEM((NP,), jnp.float32),
            pltpu.VMEM((NP,), jnp.float32),
            pltpu.VMEM((NP,), jnp.float32),
            pltpu.VMEM((2, EK), jnp.int32),
            pltpu.VMEM((2, EK), jnp.float32),
            pltpu.SemaphoreType.DMA((2,)),
        ],
    )
    return f(h_t, sd, w)


def kernel(x, edge_index, batch, W1, a_src1, a_dst1, b1,
           W2, a_src2, a_dst2, b2, Wfc, bfc):
    n = x.shape[0]
    loops = jnp.arange(n, dtype=edge_index.dtype)
    src = jnp.concatenate([edge_index[0], loops])
    dst = jnp.concatenate([edge_index[1], loops])
    src = jnp.concatenate(
        [src, jnp.zeros((EP - ETOT,), src.dtype)])
    dst = jnp.concatenate(
        [dst, jnp.full((EP - ETOT,), NP - 1, dst.dtype)])
    sd = jnp.bitwise_or(src, jnp.left_shift(dst, 16))

    x_p = jnp.pad(x, ((0, NP - n), (0, 0)))

    h1, as1, ad1 = _tc_dense(x_p, W1, a_src1, a_dst1)
    w1, den1 = _sc_pass_a(as1.reshape(NP), ad1.reshape(NP), sd)
    acc1 = _sc_pass_b(h1, sd, w1)

    h2, as2, ad2 = _tc_mid(acc1, den1, b1.reshape(H, 1), W2, a_src2, a_dst2)
    w2, den2 = _sc_pass_a(as2.reshape(NP), ad2.reshape(NP), sd)
    acc2 = _sc_pass_b(h2, sd, w2)

    batch_pad = jnp.concatenate([batch, jnp.full((NP - n,), B, batch.dtype)])

    return _tc_final(acc2, den2, b2.reshape(H, 1), batch_pad.reshape(NP, 1),
                     Wfc, bfc.reshape(1, OUT))


# speedup vs baseline: 1.4409x; 1.0788x over previous
import functools

import jax
import jax.numpy as jnp
from jax import lax
from jax.experimental import pallas as pl
from jax.experimental.pallas import tpu as pltpu
from jax.experimental.pallas import tpu_sc as plsc

N, E, DIN, H, OUT, B = 10000, 320000, 128, 128, 64, 128

NP = 10240
BLK = 512
NBLK = NP // BLK
EK = 3456
ETOT = E + N
NTILES = 32
EP = ((ETOT + NTILES * EK - 1) // (NTILES * EK)) * (NTILES * EK)
EPT = EP // NTILES
ACH = EPT // EK
NCH = EP // EK
NGR = EK // 16
FPT = H // NTILES


def _pack_pairs(hb):
    lo = lax.bitcast_convert_type(hb[:H // 2].astype(jnp.bfloat16),
                                  jnp.uint16).astype(jnp.uint32)
    hi = lax.bitcast_convert_type(hb[H // 2:].astype(jnp.bfloat16),
                                  jnp.uint16).astype(jnp.uint32)
    return lax.bitcast_convert_type(
        jnp.bitwise_or(lo, jnp.left_shift(hi, 16)), jnp.int32)


def _tc_dense_body(x_ref, w_ref, asr_ref, adr_ref, h_ref, as_ref, ad_ref):
    xb = x_ref[...]
    hb = lax.dot_general(w_ref[...], xb, (((0,), (1,)), ((), ())),
                         preferred_element_type=jnp.float32)
    h_ref[...] = _pack_pairs(hb)
    as_ref[...] = lax.dot_general(asr_ref[...], hb, (((1,), (0,)), ((), ())),
                                  preferred_element_type=jnp.float32)
    ad_ref[...] = lax.dot_general(adr_ref[...], hb, (((1,), (0,)), ((), ())),
                                  preferred_element_type=jnp.float32)


def _tc_dense(x_t, w, a_src, a_dst):
    return pl.pallas_call(
        _tc_dense_body,
        grid=(NBLK,),
        in_specs=[
            pl.BlockSpec((BLK, DIN), lambda j: (j, 0)),
            pl.BlockSpec((DIN, H), lambda j: (0, 0)),
            pl.BlockSpec((1, H), lambda j: (0, 0)),
            pl.BlockSpec((1, H), lambda j: (0, 0)),
        ],
        out_specs=[
            pl.BlockSpec((H // 2, BLK), lambda j: (0, j)),
            pl.BlockSpec((1, BLK), lambda j: (0, j)),
            pl.BlockSpec((1, BLK), lambda j: (0, j)),
        ],
        out_shape=[
            jax.ShapeDtypeStruct((H // 2, NP), jnp.int32),
            jax.ShapeDtypeStruct((1, NP), jnp.float32),
            jax.ShapeDtypeStruct((1, NP), jnp.float32),
        ],
    )(x_t, w, a_src, a_dst)


def _tc_mid_body(acc_ref, denp_ref, b_ref, w_ref, asr_ref, adr_ref,
                 h_ref, as_ref, ad_ref):
    den = jnp.sum(denp_ref[...], axis=0, keepdims=True)
    g = acc_ref[...] / (den + 1e-30) + b_ref[...]
    g = jnp.maximum(g, 0.0)
    hb = lax.dot_general(w_ref[...], g, (((0,), (0,)), ((), ())),
                         preferred_element_type=jnp.float32)
    h_ref[...] = _pack_pairs(hb)
    as_ref[...] = lax.dot_general(asr_ref[...], hb, (((1,), (0,)), ((), ())),
                                  preferred_element_type=jnp.float32)
    ad_ref[...] = lax.dot_general(adr_ref[...], hb, (((1,), (0,)), ((), ())),
                                  preferred_element_type=jnp.float32)


def _tc_mid(acc_t, den_p, b_col, w, a_src, a_dst):
    return pl.pallas_call(
        _tc_mid_body,
        grid=(NBLK,),
        in_specs=[
            pl.BlockSpec((H, BLK), lambda j: (0, j)),
            pl.BlockSpec((NTILES, BLK), lambda j: (0, j)),
            pl.BlockSpec((H, 1), lambda j: (0, 0)),
            pl.BlockSpec((H, H), lambda j: (0, 0)),
            pl.BlockSpec((1, H), lambda j: (0, 0)),
            pl.BlockSpec((1, H), lambda j: (0, 0)),
        ],
        out_specs=[
            pl.BlockSpec((H // 2, BLK), lambda j: (0, j)),
            pl.BlockSpec((1, BLK), lambda j: (0, j)),
            pl.BlockSpec((1, BLK), lambda j: (0, j)),
        ],
        out_shape=[
            jax.ShapeDtypeStruct((H // 2, NP), jnp.int32),
            jax.ShapeDtypeStruct((1, NP), jnp.float32),
            jax.ShapeDtypeStruct((1, NP), jnp.float32),
        ],
    )(acc_t, den_p, b_col, w, a_src, a_dst)


def _tc_final_body(acc_ref, denp_ref, b_ref, batch_ref, wfc_ref, bfc_ref,
                   out_ref, pooled_ref, cnt_ref):
    j = pl.program_id(0)

    @pl.when(j == 0)
    def _():
        pooled_ref[...] = jnp.zeros_like(pooled_ref)
        cnt_ref[...] = jnp.zeros_like(cnt_ref)

    den = jnp.sum(denp_ref[...], axis=0, keepdims=True)
    g = acc_ref[...] / (den + 1e-30) + b_ref[...]
    g = jnp.maximum(g, 0.0)
    ids = lax.broadcasted_iota(jnp.int32, (BLK, B), 1)
    pb = (batch_ref[...] == ids).astype(jnp.float32)
    pooled_ref[...] += jnp.dot(g, pb, preferred_element_type=jnp.float32)
    cnt_ref[...] += jnp.sum(pb, axis=0, keepdims=True)

    @pl.when(j == NBLK - 1)
    def _():
        pooled = pooled_ref[...] / jnp.maximum(cnt_ref[...], 1.0)
        out_ref[...] = lax.dot_general(
            pooled, wfc_ref[...], (((0,), (0,)), ((), ())),
            preferred_element_type=jnp.float32) + bfc_ref[...]


def _tc_final(acc_t, den_p, b_col, batch_col, wfc, bfc_row):
    return pl.pallas_call(
        _tc_final_body,
        grid=(NBLK,),
        in_specs=[
            pl.BlockSpec((H, BLK), lambda j: (0, j)),
            pl.BlockSpec((NTILES, BLK), lambda j: (0, j)),
            pl.BlockSpec((H, 1), lambda j: (0, 0)),
            pl.BlockSpec((BLK, 1), lambda j: (j, 0)),
            pl.BlockSpec((H, OUT), lambda j: (0, 0)),
            pl.BlockSpec((1, OUT), lambda j: (0, 0)),
        ],
        out_specs=pl.BlockSpec((B, OUT), lambda j: (0, 0)),
        out_shape=jax.ShapeDtypeStruct((B, OUT), jnp.float32),
        scratch_shapes=[
            pltpu.VMEM((H, B), jnp.float32),
            pltpu.VMEM((1, B), jnp.float32),
        ],
    )(acc_t, den_p, b_col, batch_col, wfc, bfc_row)


def _sc_mesh():
    return plsc.VectorSubcoreMesh(core_axis_name="c", subcore_axis_name="s",
                                  num_cores=2, num_subcores=16)


def _sc_pass_a_body(as_hbm, ad_hbm, sd_hbm, w_hbm, denp_hbm,
                    asv, adv, denv, sdbuf, wbuf):
    wid = lax.axis_index("c") * 16 + lax.axis_index("s")

    pltpu.sync_copy(as_hbm, asv)
    pltpu.sync_copy(ad_hbm, adv)

    neg = jnp.full((16,), -3.0e38, jnp.float32)

    def _mx(i, carry):
        ma, md = carry
        sl = pl.ds(i * 16, 16)
        return jnp.maximum(ma, asv[sl]), jnp.maximum(md, adv[sl])

    ma, md = lax.fori_loop(0, NP // 16, _mx, (neg, neg))

    def _hmax(v):
        idx = lax.iota(jnp.int32, 16)
        for k in (1, 2, 4, 8):
            perm = jnp.bitwise_xor(idx, k)
            g = lax.gather(
                v, perm[:, None],
                lax.GatherDimensionNumbers(
                    offset_dims=(), collapsed_slice_dims=(0,),
                    start_index_map=(0,)),
                (1,), mode=lax.GatherScatterMode.PROMISE_IN_BOUNDS)
            v = jnp.maximum(v, g)
        return v

    m_all = jnp.maximum(_hmax(ma) + _hmax(md), 0.0)

    zero = jnp.zeros((16,), jnp.float32)

    def _z(i, _):
        denv[pl.ds(i * 16, 16)] = zero
        return 0

    lax.fori_loop(0, NP // 16, _z, 0)

    for c in range(ACH):
        base = wid * EPT + c * EK
        pltpu.sync_copy(sd_hbm.at[pl.ds(base, EK)], sdbuf)

        @plsc.parallel_loop(0, EK, step=16, unroll=4)
        def _g(g):
            sl = pl.ds(g, 16)
            sd16 = sdbuf[sl]
            s16 = jnp.bitwise_and(sd16, 0xFFFF)
            d16 = lax.shift_right_logical(sd16, 16)
            z = plsc.load_gather(asv, [s16]) + plsc.load_gather(adv, [d16])
            e = jnp.where(z > 0.0, z, 0.2 * z)
            w16 = jnp.exp(e - m_all)
            wbuf[sl] = w16
            plsc.addupdate_scatter(denv, [d16], w16)
        pltpu.sync_copy(wbuf, w_hbm.at[pl.ds(base, EK)])

    pltpu.sync_copy(denv, denp_hbm.at[wid])


def _sc_pass_a(as_n, ad_n, sd):
    f = pl.kernel(
        _sc_pass_a_body,
        out_type=[
            jax.ShapeDtypeStruct((EP,), jnp.float32),
            jax.ShapeDtypeStruct((NTILES, NP), jnp.float32),
        ],
        mesh=_sc_mesh(),
        compiler_params=pltpu.CompilerParams(needs_layout_passes=False),
        scratch_types=[
            pltpu.VMEM((NP,), jnp.float32),
            pltpu.VMEM((NP,), jnp.float32),
            pltpu.VMEM((NP,), jnp.float32),
            pltpu.VMEM((EK,), jnp.int32),
            pltpu.VMEM((EK,), jnp.float32),
        ],
    )
    return f(as_n, ad_n, sd)


def _sc_pass_b_body(h_hbm, sd_hbm, w_hbm, out_hbm,
                    hp0, hp1, ar0, ar1, ar2, ar3,
                    sd0, sd1, w0, w1, sems):
    wid = lax.axis_index("c") * 16 + lax.axis_index("s")
    hprows = (hp0, hp1)
    arows = (ar0, ar1, ar2, ar3)
    sdb = (sd0, sd1)
    wb = (w0, w1)

    for i in range(2):
        pltpu.sync_copy(h_hbm.at[wid * 2 + i], hprows[i])

    zero = jnp.zeros((16,), jnp.float32)

    def _z(i, _):
        for f in range(4):
            arows[f][pl.ds(i * 16, 16)] = zero
        return 0

    lax.fori_loop(0, NP // 16, _z, 0)

    def _start(c, slot):
        base = c * EK
        pltpu.async_copy(sd_hbm.at[pl.ds(base, EK)], sdb[slot],
                         sems.at[slot])
        pltpu.async_copy(w_hbm.at[pl.ds(base, EK)], wb[slot],
                         sems.at[slot])

    def _wait(c, slot):
        base = c * EK
        pltpu.make_async_copy(sd_hbm.at[pl.ds(base, EK)], sdb[slot],
                              sems.at[slot]).wait()
        pltpu.make_async_copy(w_hbm.at[pl.ds(base, EK)], wb[slot],
                              sems.at[slot]).wait()

    def _process(slot):
        @plsc.parallel_loop(0, EK, step=16, unroll=4)
        def _g(g):
            sl = pl.ds(g, 16)
            sd16 = sdb[slot][sl]
            s16 = jnp.bitwise_and(sd16, 0xFFFF)
            d16 = lax.shift_right_logical(sd16, 16)
            w16 = wb[slot][sl]
            for p in range(2):
                vp = plsc.load_gather(hprows[p], [s16])
                bb = plsc.bitcast(vp, jnp.bfloat16)
                lo, hi = plsc.unpack(bb, format=plsc.PackFormat.INTERLEAVED)
                plsc.addupdate_scatter(arows[p], [d16], lo * w16)
                plsc.addupdate_scatter(arows[2 + p], [d16], hi * w16)

    _start(0, 0)

    def _outer(p, _):
        c0 = p * 2
        _wait(c0, 0)
        _start(c0 + 1, 1)
        _process(0)
        _wait(c0 + 1, 1)

        @pl.when(c0 + 2 < NCH)
        def _():
            _start(c0 + 2, 0)

        _process(1)
        return 0

    lax.fori_loop(0, NCH // 2, _outer, 0)

    pltpu.sync_copy(arows[0], out_hbm.at[wid * 2])
    pltpu.sync_copy(arows[1], out_hbm.at[wid * 2 + 1])
    pltpu.sync_copy(arows[2], out_hbm.at[wid * 2 + H // 2])
    pltpu.sync_copy(arows[3], out_hbm.at[wid * 2 + 1 + H // 2])


def _sc_pass_b(h_t, sd, w):
    f = pl.kernel(
        _sc_pass_b_body,
        out_type=jax.ShapeDtypeStruct((H, NP), jnp.float32),
        mesh=_sc_mesh(),
        compiler_params=pltpu.CompilerParams(needs_layout_passes=False),
        scratch_types=[
            pltpu.VMEM((NP,), jnp.int32),
            pltpu.VMEM((NP,), jnp.int32),
            pltpu.VMEM((NP,), jnp.float32),
            pltpu.VMEM((NP,), jnp.float32),
            pltpu.VMEM((NP,), jnp.float32),
            pltpu.VMEM((NP,), jnp.float32),
            pltpu.VMEM((EK,), jnp.int32),
            pltpu.VMEM((EK,), jnp.int32),
            pltpu.VMEM((EK,), jnp.float32),
            pltpu.VMEM((EK,), jnp.float32),
            pltpu.SemaphoreType.DMA((2,)),
        ],
    )
    return f(h_t, sd, w)


def kernel(x, edge_index, batch, W1, a_src1, a_dst1, b1,
           W2, a_src2, a_dst2, b2, Wfc, bfc):
    n = x.shape[0]
    loops = jnp.arange(n, dtype=edge_index.dtype)
    src = jnp.concatenate([edge_index[0], loops])
    dst = jnp.concatenate([edge_index[1], loops])
    src = jnp.concatenate(
        [src, jnp.zeros((EP - ETOT,), src.dtype)])
    dst = jnp.concatenate(
        [dst, jnp.full((EP - ETOT,), NP - 1, dst.dtype)])
    sd = jnp.bitwise_or(src, jnp.left_shift(dst, 16))

    x_p = jnp.pad(x, ((0, NP - n), (0, 0)))

    h1, as1, ad1 = _tc_dense(x_p, W1, a_src1, a_dst1)
    w1, den1 = _sc_pass_a(as1.reshape(NP), ad1.reshape(NP), sd)
    acc1 = _sc_pass_b(h1, sd, w1)

    h2, as2, ad2 = _tc_mid(acc1, den1, b1.reshape(H, 1), W2, a_src2, a_dst2)
    w2, den2 = _sc_pass_a(as2.reshape(NP), ad2.reshape(NP), sd)
    acc2 = _sc_pass_b(h2, sd, w2)

    batch_pad = jnp.concatenate([batch, jnp.full((NP - n,), B, batch.dtype)])

    return _tc_final(acc2, den2, b2.reshape(H, 1), batch_pad.reshape(NP, 1),
                     Wfc, bfc.reshape(1, OUT))
